# baseline (speedup 1.0000x reference)
import numpy as np
import jax
import jax.numpy as jnp
from jax.experimental import pallas as pl

_TOKENS = 8192
_D = 32
_E = 4
_BLK = 1024

_CONTRACT_1_1 = (((1,), (1,)), ((), ()))
_CONTRACT_1_0 = (((1,), (0,)), ((), ()))

_PAIRS = [(e, f) for e in range(_E) for f in range(e + 1, _E)]

_DIF = np.zeros((_E, len(_PAIRS)), np.float32)
_M = np.zeros((len(_PAIRS), _E), np.float32)
for p, (e, f) in enumerate(_PAIRS):
    _DIF[f, p] = 1.0
    _DIF[e, p] = -1.0
    _M[p, e] = 1.0
    _M[p, f] = -1.0
_BCAST = np.zeros((_E, _E * _D), np.float32)
_SUM = np.zeros((_E * _D, _D), np.float32)
for e in range(_E):
    for o in range(_D):
        _BCAST[e, e * _D + o] = 1.0
        _SUM[e * _D + o, o] = 1.0


def _moe_block(x_ref, gw_ref, wcat_ref, dif_ref, m_ref, bcast_ref, sum_ref,
               out_ref):
    xb = x_ref[...]
    logits = jax.lax.dot_general(xb, gw_ref[...], _CONTRACT_1_1,
                                 preferred_element_type=jnp.float32)
    y = jax.lax.dot_general(xb, wcat_ref[...], _CONTRACT_1_1,
                            preferred_element_type=jnp.float32)
    d = jax.lax.dot_general(logits, dif_ref[...], _CONTRACT_1_0,
                            precision=jax.lax.Precision.HIGHEST,
                            preferred_element_type=jnp.float32)
    c = (d > 0).astype(jnp.float32)
    bm = jax.lax.dot_general(c, m_ref[...], _CONTRACT_1_0,
                             preferred_element_type=jnp.float32)
    lane = jax.lax.broadcasted_iota(jnp.int32, bm.shape, 1).astype(jnp.float32)
    mask = (bm + lane < 1.5).astype(jnp.float32)
    maskf = jax.lax.dot_general(mask, bcast_ref[...], _CONTRACT_1_0,
                                preferred_element_type=jnp.float32)
    out_ref[...] = jax.lax.dot_general(maskf * y, sum_ref[...], _CONTRACT_1_0,
                                       precision=jax.lax.Precision.HIGHEST,
                                       preferred_element_type=jnp.float32)


@jax.jit
def kernel(x, gate_w, expert_w):
    wcat = expert_w.reshape(_E * _D, _D)
    grid = (_TOKENS // _BLK,)
    full = lambda a: pl.BlockSpec(a.shape, lambda i: (0,) * a.ndim)
    consts = (jnp.asarray(_DIF), jnp.asarray(_M), jnp.asarray(_BCAST),
              jnp.asarray(_SUM))
    return pl.pallas_call(
        _moe_block,
        grid=grid,
        in_specs=[
            pl.BlockSpec((_BLK, _D), lambda i: (i, 0)),
            full(gate_w), full(wcat),
            *[full(c) for c in consts],
        ],
        out_specs=pl.BlockSpec((_BLK, _D), lambda i: (i, 0)),
        out_shape=jax.ShapeDtypeStruct((_TOKENS, _D), jnp.float32),
    )(x, gate_w, wcat, *consts)


# speedup vs baseline: 1.3448x; 1.3448x over previous
import numpy as np
import jax
import jax.numpy as jnp
from jax.experimental import pallas as pl

_TOKENS = 8192
_D = 32
_E = 4
_BLK = 1024

_CONTRACT_1_1 = (((1,), (1,)), ((), ()))
_CONTRACT_1_0 = (((1,), (0,)), ((), ()))

_PAIRS = [(e, f) for e in range(_E) for f in range(e + 1, _E)]

_DIF = np.zeros((_E, len(_PAIRS)), np.float32)
_M = np.zeros((len(_PAIRS), _E), np.float32)
for p, (e, f) in enumerate(_PAIRS):
    _DIF[f, p] = 1.0
    _DIF[e, p] = -1.0
    _M[p, e] = 1.0
    _M[p, f] = -1.0
_BCAST = np.zeros((_E, _E * _D), np.float32)
_SUM = np.zeros((_E * _D, _D), np.float32)
for e in range(_E):
    for o in range(_D):
        _BCAST[e, e * _D + o] = 1.0
        _SUM[e * _D + o, o] = 1.0


def _moe_block(x_ref, gw_ref, wcat_ref, dif_ref, m_ref, bcast_ref, out_ref):
    xb = x_ref[...]
    logits = jax.lax.dot_general(xb, gw_ref[...], _CONTRACT_1_1,
                                 preferred_element_type=jnp.float32)
    y = jax.lax.dot_general(xb, wcat_ref[...], _CONTRACT_1_1,
                            preferred_element_type=jnp.float32)
    d = jax.lax.dot_general(logits, dif_ref[...], _CONTRACT_1_0,
                            precision=jax.lax.Precision.HIGHEST,
                            preferred_element_type=jnp.float32)
    c = (d > 0).astype(jnp.float32)
    bm = jax.lax.dot_general(c, m_ref[...], _CONTRACT_1_0,
                             preferred_element_type=jnp.float32)
    lane = jax.lax.broadcasted_iota(jnp.int32, bm.shape, 1).astype(jnp.float32)
    mask = (bm + lane < 1.5).astype(jnp.float32)
    maskf = jax.lax.dot_general(mask, bcast_ref[...], _CONTRACT_1_0,
                                preferred_element_type=jnp.float32)
    p = maskf * y
    out_ref[...] = ((p[:, 0 * _D:1 * _D] + p[:, 1 * _D:2 * _D]) +
                    (p[:, 2 * _D:3 * _D] + p[:, 3 * _D:4 * _D]))


@jax.jit
def kernel(x, gate_w, expert_w):
    wcat = expert_w.reshape(_E * _D, _D)
    grid = (_TOKENS // _BLK,)
    full = lambda a: pl.BlockSpec(a.shape, lambda i: (0,) * a.ndim)
    consts = (jnp.asarray(_DIF), jnp.asarray(_M), jnp.asarray(_BCAST))
    return pl.pallas_call(
        _moe_block,
        grid=grid,
        in_specs=[
            pl.BlockSpec((_BLK, _D), lambda i: (i, 0)),
            full(gate_w), full(wcat),
            *[full(c) for c in consts],
        ],
        out_specs=pl.BlockSpec((_BLK, _D), lambda i: (i, 0)),
        out_shape=jax.ShapeDtypeStruct((_TOKENS, _D), jnp.float32),
    )(x, gate_w, wcat, *consts)


# speedup vs baseline: 1.5560x; 1.1570x over previous
import numpy as np
import jax
import jax.numpy as jnp
from jax.experimental import pallas as pl

_TOKENS = 8192
_D = 32
_E = 4
_BLK = 1024

_CONTRACT_1_1 = (((1,), (1,)), ((), ()))
_CONTRACT_1_0 = (((1,), (0,)), ((), ()))

_PAIRS = [(e, f) for e in range(_E) for f in range(e + 1, _E)]
_NP = len(_PAIRS)

_M = np.zeros((_NP, _E), np.float32)
for p, (e, f) in enumerate(_PAIRS):
    _M[p, e] = 1.0
    _M[p, f] = -1.0
_BCAST = np.zeros((_E, _E * _D), np.float32)
for e in range(_E):
    _BCAST[e, e * _D:(e + 1) * _D] = 1.0


def _moe_block(x_ref, gw_ref, wcat_ref, m_ref, bcast_ref, out_ref):
    xb = x_ref[...]
    gw = gw_ref[...]
    gpq = jnp.concatenate(
        [gw[e:e + 1] for e, _ in _PAIRS] + [gw[f:f + 1] for _, f in _PAIRS],
        axis=0)
    pq = jax.lax.dot_general(xb, gpq, _CONTRACT_1_1,
                             preferred_element_type=jnp.float32)
    c = (pq[:, _NP:2 * _NP] > pq[:, 0:_NP]).astype(jnp.float32)
    bm = jax.lax.dot_general(c, m_ref[...], _CONTRACT_1_0,
                             preferred_element_type=jnp.float32)
    lane = jax.lax.broadcasted_iota(jnp.int32, bm.shape, 1).astype(jnp.float32)
    mask = (bm + lane < 1.5).astype(jnp.float32)
    maskf = jax.lax.dot_general(mask, bcast_ref[...], _CONTRACT_1_0,
                                preferred_element_type=jnp.float32)
    y = jax.lax.dot_general(xb, wcat_ref[...], _CONTRACT_1_1,
                            preferred_element_type=jnp.float32)
    p = maskf * y
    out_ref[...] = ((p[:, 0 * _D:1 * _D] + p[:, 1 * _D:2 * _D]) +
                    (p[:, 2 * _D:3 * _D] + p[:, 3 * _D:4 * _D]))


@jax.jit
def kernel(x, gate_w, expert_w):
    wcat = expert_w.reshape(_E * _D, _D)
    grid = (_TOKENS // _BLK,)
    full = lambda a: pl.BlockSpec(a.shape, lambda i: (0,) * a.ndim)
    consts = (jnp.asarray(_M), jnp.asarray(_BCAST))
    return pl.pallas_call(
        _moe_block,
        grid=grid,
        in_specs=[
            pl.BlockSpec((_BLK, _D), lambda i: (i, 0)),
            full(gate_w), full(wcat),
            *[full(c) for c in consts],
        ],
        out_specs=pl.BlockSpec((_BLK, _D), lambda i: (i, 0)),
        out_shape=jax.ShapeDtypeStruct((_TOKENS, _D), jnp.float32),
    )(x, gate_w, wcat, *consts)


# speedup vs baseline: 1.7775x; 1.1423x over previous
import numpy as np
import jax
import jax.numpy as jnp
from jax.experimental import pallas as pl

_TOKENS = 8192
_D = 32
_E = 4
_BLK = 2048

_CONTRACT_1_1 = (((1,), (1,)), ((), ()))
_CONTRACT_1_0 = (((1,), (0,)), ((), ()))

_PAIRS = [(e, f) for e in range(_E) for f in range(e + 1, _E)]
_NP = len(_PAIRS)

_M = np.zeros((_NP, _E), np.float32)
for p, (e, f) in enumerate(_PAIRS):
    _M[p, e] = 1.0
    _M[p, f] = -1.0
_BCAST = np.zeros((_E, _E * _D), np.float32)
for e in range(_E):
    _BCAST[e, e * _D:(e + 1) * _D] = 1.0


def _moe_block(x_ref, gw_ref, wcat_ref, m_ref, bcast_ref, out_ref):
    xb = x_ref[...]
    gw = gw_ref[...]
    gpq = jnp.concatenate(
        [gw[e:e + 1] for e, _ in _PAIRS] + [gw[f:f + 1] for _, f in _PAIRS],
        axis=0)
    pq = jax.lax.dot_general(xb, gpq, _CONTRACT_1_1,
                             preferred_element_type=jnp.float32)
    c = (pq[:, _NP:2 * _NP] > pq[:, 0:_NP]).astype(jnp.float32)
    bm = jax.lax.dot_general(c, m_ref[...], _CONTRACT_1_0,
                             preferred_element_type=jnp.float32)
    lane = jax.lax.broadcasted_iota(jnp.int32, bm.shape, 1).astype(jnp.float32)
    mask = (bm + lane < 1.5).astype(jnp.float32)
    maskf = jax.lax.dot_general(mask, bcast_ref[...], _CONTRACT_1_0,
                                preferred_element_type=jnp.float32)
    y = jax.lax.dot_general(xb, wcat_ref[...], _CONTRACT_1_1,
                            preferred_element_type=jnp.float32)
    p = maskf * y
    out_ref[...] = ((p[:, 0 * _D:1 * _D] + p[:, 1 * _D:2 * _D]) +
                    (p[:, 2 * _D:3 * _D] + p[:, 3 * _D:4 * _D]))


@jax.jit
def kernel(x, gate_w, expert_w):
    wcat = expert_w.reshape(_E * _D, _D)
    grid = (_TOKENS // _BLK,)
    full = lambda a: pl.BlockSpec(a.shape, lambda i: (0,) * a.ndim)
    consts = (jnp.asarray(_M), jnp.asarray(_BCAST))
    return pl.pallas_call(
        _moe_block,
        grid=grid,
        in_specs=[
            pl.BlockSpec((_BLK, _D), lambda i: (i, 0)),
            full(gate_w), full(wcat),
            *[full(c) for c in consts],
        ],
        out_specs=pl.BlockSpec((_BLK, _D), lambda i: (i, 0)),
        out_shape=jax.ShapeDtypeStruct((_TOKENS, _D), jnp.float32),
    )(x, gate_w, wcat, *consts)


# speedup vs baseline: 1.8075x; 1.0169x over previous
import numpy as np
import jax
import jax.numpy as jnp
from jax.experimental import pallas as pl

_TOKENS = 8192
_D = 32
_E = 4
_BLK = 4096

_CONTRACT_1_1 = (((1,), (1,)), ((), ()))
_CONTRACT_1_0 = (((1,), (0,)), ((), ()))

_PAIRS = [(e, f) for e in range(_E) for f in range(e + 1, _E)]
_NP = len(_PAIRS)

_M = np.zeros((_NP, _E), np.float32)
for p, (e, f) in enumerate(_PAIRS):
    _M[p, e] = 1.0
    _M[p, f] = -1.0
_BCAST = np.zeros((_E, _E * _D), np.float32)
for e in range(_E):
    _BCAST[e, e * _D:(e + 1) * _D] = 1.0


def _moe_block(x_ref, gw_ref, wcat_ref, m_ref, bcast_ref, out_ref):
    xb = x_ref[...]
    gw = gw_ref[...]
    gpq = jnp.concatenate(
        [gw[e:e + 1] for e, _ in _PAIRS] + [gw[f:f + 1] for _, f in _PAIRS],
        axis=0)
    pq = jax.lax.dot_general(xb, gpq, _CONTRACT_1_1,
                             preferred_element_type=jnp.float32)
    c = (pq[:, _NP:2 * _NP] > pq[:, 0:_NP]).astype(jnp.float32)
    bm = jax.lax.dot_general(c, m_ref[...], _CONTRACT_1_0,
                             preferred_element_type=jnp.float32)
    lane = jax.lax.broadcasted_iota(jnp.int32, bm.shape, 1).astype(jnp.float32)
    mask = (bm + lane < 1.5).astype(jnp.float32)
    maskf = jax.lax.dot_general(mask, bcast_ref[...], _CONTRACT_1_0,
                                preferred_element_type=jnp.float32)
    y = jax.lax.dot_general(xb, wcat_ref[...], _CONTRACT_1_1,
                            preferred_element_type=jnp.float32)
    p = maskf * y
    out_ref[...] = ((p[:, 0 * _D:1 * _D] + p[:, 1 * _D:2 * _D]) +
                    (p[:, 2 * _D:3 * _D] + p[:, 3 * _D:4 * _D]))


@jax.jit
def kernel(x, gate_w, expert_w):
    wcat = expert_w.reshape(_E * _D, _D)
    grid = (_TOKENS // _BLK,)
    full = lambda a: pl.BlockSpec(a.shape, lambda i: (0,) * a.ndim)
    consts = (jnp.asarray(_M), jnp.asarray(_BCAST))
    return pl.pallas_call(
        _moe_block,
        grid=grid,
        in_specs=[
            pl.BlockSpec((_BLK, _D), lambda i: (i, 0)),
            full(gate_w), full(wcat),
            *[full(c) for c in consts],
        ],
        out_specs=pl.BlockSpec((_BLK, _D), lambda i: (i, 0)),
        out_shape=jax.ShapeDtypeStruct((_TOKENS, _D), jnp.float32),
    )(x, gate_w, wcat, *consts)
